# baseline (speedup 1.0000x reference)
import functools

import jax
import jax.numpy as jnp
from jax import lax
from jax.experimental import pallas as pl
from jax.experimental.pallas import tpu as pltpu
from jax.experimental.pallas import tpu_sc as plsc

N = 10000
E = 320000
D = 128
H = 128
C = 3
XW = 16
N_PAD = 10240

NC = 2
NS = 16
NW = NC * NS
PER_W = E // NW
RPT = N_PAD // NS

NCHK = 5
EC = E // NCHK
PWC = EC // NW

BN = 1000
BE = 1000

_f32 = jnp.float32


def _mesh():
    return plsc.VectorSubcoreMesh(core_axis_name="c", subcore_axis_name="s",
                                  num_cores=NC, num_subcores=NS)


def _table_body(h_ref, wr_ref, wc_ref, tr_ref, tc_ref):
    h = h_ref[...]
    tr_ref[...] = jnp.dot(h, wr_ref[...], preferred_element_type=_f32)
    tc_ref[...] = jnp.dot(h, wc_ref[...], preferred_element_type=_f32)


_table_call = pl.pallas_call(
    _table_body,
    grid=(N // BN,),
    in_specs=[
        pl.BlockSpec((BN, D), lambda i: (i, 0)),
        pl.BlockSpec((D, D), lambda i: (0, 0)),
        pl.BlockSpec((D, D), lambda i: (0, 0)),
    ],
    out_specs=[
        pl.BlockSpec((BN, D), lambda i: (i, 0)),
        pl.BlockSpec((BN, D), lambda i: (i, 0)),
    ],
    out_shape=[
        jax.ShapeDtypeStruct((N, D), _f32),
        jax.ShapeDtypeStruct((N, D), _f32),
    ],
)


KG = 80
NBUF = 5


@functools.cache
def _sc_gather_call(ci):
    @functools.partial(
        pl.kernel,
        out_type=jax.ShapeDtypeStruct((EC, D), _f32),
        mesh=_mesh(),
        scratch_types=[
            pltpu.VMEM((PWC,), jnp.int32),
            pltpu.VMEM((PWC,), jnp.int32),
            [pltpu.VMEM((KG, D), _f32) for _ in range(NBUF)],
            [pltpu.VMEM((KG, D), _f32) for _ in range(NBUF)],
            pltpu.SemaphoreType.DMA,
            pltpu.SemaphoreType.DMA,
        ],
        compiler_params=pltpu.CompilerParams(use_tc_tiling_on_sc=True),
    )
    def _sc_gather(tr_hbm, tc_hbm, row_hbm, col_hbm, g_hbm,
                   ir_v, ic_v, bufr, bufc, gsem, wsem):
        wid = lax.axis_index("s") * NC + lax.axis_index("c")
        tile_base = wid * PWC
        pltpu.sync_copy(row_hbm.at[pl.ds(ci * EC + tile_base, PWC)], ir_v)
        pltpu.sync_copy(col_hbm.at[pl.ds(ci * EC + tile_base, PWC)], ic_v)

        def rnd(r, carry2):
            rbase = r * (NBUF * KG)
            gs = []
            for q in range(NBUF):
                isl = pl.ds(rbase + q * KG, KG)
                gs.append((
                    pltpu.async_copy(tr_hbm.at[ir_v.at[isl]], bufr[q], gsem),
                    pltpu.async_copy(tc_hbm.at[ic_v.at[isl]], bufc[q], gsem),
                ))
            ws = []
            for q in range(NBUF):
                gs[q][0].wait()
                gs[q][1].wait()

                def add_row(i, carry3, q=q):
                    for j in range(D // 16):
                        sl = pl.ds(j * 16, 16)
                        bufr[q][i, sl] = bufr[q][i, sl] + bufc[q][i, sl]
                    return carry3

                lax.fori_loop(0, KG, add_row, 0)
                ws.append(pltpu.async_copy(
                    bufr[q],
                    g_hbm.at[pl.ds(tile_base + rbase + q * KG, KG)],
                    wsem))
            for w in ws:
                w.wait()
            return carry2

        lax.fori_loop(0, PWC // (NBUF * KG), rnd, 0)

    return _sc_gather


KX = 400


@functools.cache
def _sc_xgather_call():
    @functools.partial(
        pl.kernel,
        out_type=jax.ShapeDtypeStruct((E, XW), _f32),
        mesh=_mesh(),
        scratch_types=[
            pltpu.VMEM((PER_W,), jnp.int32),
            pltpu.VMEM((PER_W,), jnp.int32),
            [pltpu.VMEM((KX, XW), _f32) for _ in range(NBUF)],
            [pltpu.VMEM((KX, XW), _f32) for _ in range(NBUF)],
            pltpu.SemaphoreType.DMA,
            pltpu.SemaphoreType.DMA,
        ],
        compiler_params=pltpu.CompilerParams(use_tc_tiling_on_sc=False),
    )
    def _sc_xgather(xp_hbm, idx_hbm, rx_hbm,
                    ir_v, ic_v, bufr, bufc, gsem, wsem):
        wid = lax.axis_index("s") * NC + lax.axis_index("c")
        tile_base = wid * PER_W
        pltpu.sync_copy(idx_hbm.at[0, pl.ds(tile_base, PER_W)], ir_v)
        pltpu.sync_copy(idx_hbm.at[1, pl.ds(tile_base, PER_W)], ic_v)

        def rnd(r, carry2):
            rbase = r * (NBUF * KX)
            gs = []
            for q in range(NBUF):
                isl = pl.ds(rbase + q * KX, KX)
                gs.append((
                    pltpu.async_copy(xp_hbm.at[ir_v.at[isl]], bufr[q], gsem),
                    pltpu.async_copy(xp_hbm.at[ic_v.at[isl]], bufc[q], gsem),
                ))
            ws = []
            for q in range(NBUF):
                gs[q][0].wait()
                gs[q][1].wait()

                def sub_row(i, carry3, q=q):
                    bufr[q][i, :] = bufr[q][i, :] - bufc[q][i, :]
                    return carry3

                lax.fori_loop(0, KX, sub_row, 0)
                ws.append(pltpu.async_copy(
                    bufr[q],
                    rx_hbm.at[pl.ds(tile_base + rbase + q * KX, KX)],
                    wsem))
            for w in ws:
                w.wait()
            return carry2

        lax.fori_loop(0, PER_W // (NBUF * KX), rnd, 0)

    return _sc_xgather


def _edge_body(s_ref, sx_ref, ea_ref,
               w1a_ref, w1d_ref, b1_ref, w2_ref, b2_ref, w3_ref, b3_ref,
               aw1_ref, ab1_ref, aw2_ref, ab2_ref,
               cw1_ref, cb1_ref, cw2_ref, cb2_ref, cw3_ref, cb3_ref,
               m_ref, cv_ref):
    silu = jax.nn.silu
    rel = sx_ref[:, :C]
    dist = jnp.sqrt(jnp.sum(rel * rel, axis=-1, keepdims=True) + 1e-12)
    pre1 = (jnp.dot(ea_ref[...], w1a_ref[...], preferred_element_type=_f32)
            + s_ref[...] + dist * w1d_ref[...] + b1_ref[...])
    m = silu(pre1)
    m = silu(jnp.dot(m, w2_ref[...], preferred_element_type=_f32) + b2_ref[...])
    m = jnp.dot(m, w3_ref[...], preferred_element_type=_f32) + b3_ref[...]
    a = silu(jnp.dot(m, aw1_ref[...], preferred_element_type=_f32) + ab1_ref[...])
    att = jax.nn.sigmoid(
        jnp.dot(a, aw2_ref[...], preferred_element_type=_f32) + ab2_ref[...])
    m = m * att
    c = silu(jnp.dot(m, cw1_ref[...], preferred_element_type=_f32) + cb1_ref[...])
    c = silu(jnp.dot(c, cw2_ref[...], preferred_element_type=_f32) + cb2_ref[...])
    co = jnp.dot(c, cw3_ref[...], preferred_element_type=_f32) + cb3_ref[...]
    cvec = co * rel / (dist + 1e-08)
    m_ref[...] = m
    cv_ref[...] = jnp.concatenate(
        [cvec, jnp.zeros((BE, XW - C), _f32)], axis=1)


_full = lambda r, c: pl.BlockSpec((r, c), lambda i: (0, 0))
NBE = EC // BE


@functools.cache
def _edge_call(ci):
    off = ci * NBE
    return pl.pallas_call(
        _edge_body,
        grid=(NBE,),
        in_specs=[
            pl.BlockSpec((BE, D), lambda i: (i, 0)),
            pl.BlockSpec((BE, XW), lambda i: (off + i, 0)),
            pl.BlockSpec((BE, 4), lambda i: (off + i, 0)),
            _full(4, H), _full(1, H), _full(1, H),
            _full(H, H), _full(1, H), _full(H, H), _full(1, H),
            _full(H, H), _full(1, H), _full(H, 1), _full(1, 1),
            _full(H, H), _full(1, H), _full(H, H), _full(1, H),
            _full(H, C), _full(1, C),
        ],
        out_specs=[
            pl.BlockSpec((BE, D), lambda i: (i, 0)),
            pl.BlockSpec((BE, XW), lambda i: (i, 0)),
        ],
        out_shape=[
            jax.ShapeDtypeStruct((EC, D), _f32),
            jax.ShapeDtypeStruct((EC, XW), _f32),
        ],
    )


KSC = 200


@functools.cache
def _sc_scatter_call(cis):
    nin = len(cis)
    @functools.partial(
        pl.kernel,
        out_type=jax.ShapeDtypeStruct((NC, N_PAD, D), _f32),
        mesh=_mesh(),
        scratch_types=[
            pltpu.VMEM((KSC,), jnp.int32),
            pltpu.VMEM((KSC, D), _f32),
            pltpu.VMEM_SHARED((N_PAD, D), _f32),
            pltpu.SemaphoreType.DMA,
        ],
        compiler_params=pltpu.CompilerParams(use_tc_tiling_on_sc=True),
    )
    def _sc_scatter(*args):
        ms, (row_hbm, zero_hbm, out_hbm, idx_v, mbuf_v, acc_sh, sem) = (
            args[:nin], args[nin:])
        cid = lax.axis_index("c")
        sid = lax.axis_index("s")
        wid = sid * NC + cid
        pltpu.sync_copy(zero_hbm.at[pl.ds(sid * RPT, RPT)],
                        acc_sh.at[pl.ds(sid * RPT, RPT)])
        plsc.subcore_barrier()

        for ci, mc in zip(cis, ms):
            def body(i, carry, mc=mc, ci=ci):
                lbase = wid * PWC + i * KSC
                pltpu.sync_copy(row_hbm.at[pl.ds(ci * EC + lbase, KSC)],
                                idx_v)
                pltpu.sync_copy(mc.at[pl.ds(lbase, KSC)], mbuf_v)
                pltpu.sync_copy(mbuf_v, acc_sh.at[idx_v], add=True)
                return carry

            lax.fori_loop(0, PWC // KSC, body, 0)
        plsc.subcore_barrier()
        pltpu.sync_copy(acc_sh.at[pl.ds(sid * RPT, RPT)],
                        out_hbm.at[cid, pl.ds(sid * RPT, RPT)])

    return _sc_scatter


KXS = 1000


@functools.cache
def _sc_xscatter_call():
    @functools.partial(
        pl.kernel,
        out_type=jax.ShapeDtypeStruct((NC, N_PAD, XW), _f32),
        mesh=_mesh(),
        scratch_types=[
            pltpu.VMEM((KXS,), jnp.int32),
            pltpu.VMEM((KXS, XW), _f32),
            pltpu.VMEM_SHARED((N_PAD, XW), _f32),
            pltpu.SemaphoreType.DMA,
        ],
        compiler_params=pltpu.CompilerParams(use_tc_tiling_on_sc=False),
    )
    def _sc_xscatter(c0, c1, c2, c3, c4, idx_hbm, zero_hbm, out_hbm,
                     idx_v, cbuf_v, acc_sh, sem):
        cid = lax.axis_index("c")
        sid = lax.axis_index("s")
        wid = sid * NC + cid
        pltpu.sync_copy(zero_hbm.at[pl.ds(sid * RPT, RPT)],
                        acc_sh.at[pl.ds(sid * RPT, RPT)])
        plsc.subcore_barrier()

        for ci, cc in enumerate((c0, c1, c2, c3, c4)):
            def body(i, carry, cc=cc, ci=ci):
                lbase = wid * PWC + i * KXS
                pltpu.sync_copy(idx_hbm.at[0, pl.ds(ci * EC + lbase, KXS)],
                                idx_v)
                pltpu.sync_copy(cc.at[pl.ds(lbase, KXS)], cbuf_v)
                pltpu.sync_copy(cbuf_v, acc_sh.at[idx_v], add=True)
                return carry

            lax.fori_loop(0, PWC // KXS, body, 0)
        plsc.subcore_barrier()
        pltpu.sync_copy(acc_sh.at[pl.ds(sid * RPT, RPT)],
                        out_hbm.at[cid, pl.ds(sid * RPT, RPT)])

    return _sc_xscatter


def _node_body(h_ref, x_ref, p_ref, pb_ref, px_ref,
               w1h_ref, w1a_ref, b1_ref, w2_ref, b2_ref, w3_ref, b3_ref,
               ho_ref, xo_ref):
    silu = jax.nn.silu
    h = h_ref[...]
    aggr = p_ref[0] + p_ref[1] + pb_ref[0] + pb_ref[1]
    caggr = px_ref[0, :, :C] + px_ref[1, :, :C]
    t = silu(jnp.dot(h, w1h_ref[...], preferred_element_type=_f32)
             + jnp.dot(aggr, w1a_ref[...], preferred_element_type=_f32)
             + b1_ref[...])
    t = silu(jnp.dot(t, w2_ref[...], preferred_element_type=_f32) + b2_ref[...])
    ho_ref[...] = h + jnp.dot(t, w3_ref[...],
                              preferred_element_type=_f32) + b3_ref[...]
    xo_ref[...] = x_ref[...] + caggr


_node_call = pl.pallas_call(
    _node_body,
    grid=(N // BN,),
    in_specs=[
        pl.BlockSpec((BN, D), lambda i: (i, 0)),
        pl.BlockSpec((BN, C), lambda i: (i, 0)),
        pl.BlockSpec((NC, BN, D), lambda i: (0, i, 0)),
        pl.BlockSpec((NC, BN, D), lambda i: (0, i, 0)),
        pl.BlockSpec((NC, BN, XW), lambda i: (0, i, 0)),
        _full(D, H), _full(H, H), _full(1, H),
        _full(H, H), _full(1, H), _full(H, D), _full(1, D),
    ],
    out_specs=[
        pl.BlockSpec((BN, D), lambda i: (i, 0)),
        pl.BlockSpec((BN, C), lambda i: (i, 0)),
    ],
    out_shape=[
        jax.ShapeDtypeStruct((N, D), _f32),
        jax.ShapeDtypeStruct((N, C), _f32),
    ],
)


def kernel(h, x, edge_index, edge_attr,
           ew1, eb1, ew2, eb2, ew3, eb3,
           nw1, nb1, nw2, nb2, nw3, nb3,
           cw1, cb1, cw2, cb2, cw3, cb3,
           aw1, ab1, aw2, ab2):
    eidx = edge_index.astype(jnp.int32)
    row = eidx[0]
    col = eidx[1]
    xp = jnp.pad(x, ((0, 0), (0, XW - C)))

    t_r, t_c = _table_call(h, ew1[4:4 + D], ew1[4 + D:4 + 2 * D])
    relx = _sc_xgather_call()(xp, eidx)
    edge_w = (ew1[:4], ew1[4 + 2 * D:].reshape(1, H), eb1.reshape(1, H),
              ew2, eb2.reshape(1, H), ew3, eb3.reshape(1, H),
              aw1, ab1.reshape(1, H), aw2, ab2.reshape(1, 1),
              cw1, cb1.reshape(1, H), cw2, cb2.reshape(1, H),
              cw3, cb3.reshape(1, C))
    ms, cvs = [], []
    for ci in range(NCHK):
        s_c = _sc_gather_call(ci)(t_r, t_c, row, col)
        m_c, cv_c = _edge_call(ci)(s_c, relx, edge_attr, *edge_w)
        ms.append(m_c)
        cvs.append(cv_c)
    zeros_m = jnp.zeros((N_PAD, D), _f32)
    zeros_x = jnp.zeros((N_PAD, XW), _f32)
    partials_a = _sc_scatter_call((0, 1, 2, 3))(*ms[:4], row, zeros_m)
    partials_b = _sc_scatter_call((4,))(ms[4], row, zeros_m)
    partials_x = _sc_xscatter_call()(*cvs, eidx, zeros_x)
    h_out, x_out = _node_call(
        h, x, partials_a, partials_b, partials_x,
        nw1[:D], nw1[D:], nb1.reshape(1, H),
        nw2, nb2.reshape(1, H), nw3, nb3.reshape(1, D))
    return (h_out, x_out)


# speedup vs baseline: 1.1363x; 1.1363x over previous
import functools

import jax
import jax.numpy as jnp
from jax import lax
from jax.experimental import pallas as pl
from jax.experimental.pallas import tpu as pltpu
from jax.experimental.pallas import tpu_sc as plsc

N = 10000
E = 320000
D = 128
H = 128
C = 3
XW = 16
N_PAD = 10240

NC = 2
NS = 16
NW = NC * NS
PER_W = E // NW
RPT = N_PAD // NS

NCHK = 5
EC = E // NCHK
PWC = EC // NW

BN = 1000
BE = 2000

_f32 = jnp.float32


def _mesh():
    return plsc.VectorSubcoreMesh(core_axis_name="c", subcore_axis_name="s",
                                  num_cores=NC, num_subcores=NS)


def _table_body(h_ref, wr_ref, wc_ref, tr_ref, tc_ref):
    h = h_ref[...]
    tr_ref[...] = jnp.dot(h, wr_ref[...], preferred_element_type=_f32)
    tc_ref[...] = jnp.dot(h, wc_ref[...], preferred_element_type=_f32)


_table_call = pl.pallas_call(
    _table_body,
    grid=(N // BN,),
    in_specs=[
        pl.BlockSpec((BN, D), lambda i: (i, 0)),
        pl.BlockSpec((D, D), lambda i: (0, 0)),
        pl.BlockSpec((D, D), lambda i: (0, 0)),
    ],
    out_specs=[
        pl.BlockSpec((BN, D), lambda i: (i, 0)),
        pl.BlockSpec((BN, D), lambda i: (i, 0)),
    ],
    out_shape=[
        jax.ShapeDtypeStruct((N, D), _f32),
        jax.ShapeDtypeStruct((N, D), _f32),
    ],
)


KG = 80
NBUF = 5


@functools.cache
def _sc_gather_call(ci):
    @functools.partial(
        pl.kernel,
        out_type=jax.ShapeDtypeStruct((EC, D), _f32),
        mesh=_mesh(),
        scratch_types=[
            pltpu.VMEM((PWC,), jnp.int32),
            pltpu.VMEM((PWC,), jnp.int32),
            [pltpu.VMEM((KG, D), _f32) for _ in range(NBUF)],
            [pltpu.VMEM((KG, D), _f32) for _ in range(NBUF)],
            pltpu.SemaphoreType.DMA,
            pltpu.SemaphoreType.DMA,
        ],
        compiler_params=pltpu.CompilerParams(use_tc_tiling_on_sc=True),
    )
    def _sc_gather(tr_hbm, tc_hbm, row_hbm, col_hbm, g_hbm,
                   ir_v, ic_v, bufr, bufc, gsem, wsem):
        wid = lax.axis_index("s") * NC + lax.axis_index("c")
        tile_base = wid * PWC
        pltpu.sync_copy(row_hbm.at[pl.ds(ci * EC + tile_base, PWC)], ir_v)
        pltpu.sync_copy(col_hbm.at[pl.ds(ci * EC + tile_base, PWC)], ic_v)

        def rnd(r, carry2):
            rbase = r * (NBUF * KG)
            gs = []
            for q in range(NBUF):
                isl = pl.ds(rbase + q * KG, KG)
                gs.append((
                    pltpu.async_copy(tr_hbm.at[ir_v.at[isl]], bufr[q], gsem),
                    pltpu.async_copy(tc_hbm.at[ic_v.at[isl]], bufc[q], gsem),
                ))
            ws = []
            for q in range(NBUF):
                gs[q][0].wait()
                gs[q][1].wait()

                def add_row(i, carry3, q=q):
                    for j in range(D // 16):
                        sl = pl.ds(j * 16, 16)
                        bufr[q][i, sl] = bufr[q][i, sl] + bufc[q][i, sl]
                    return carry3

                lax.fori_loop(0, KG, add_row, 0)
                ws.append(pltpu.async_copy(
                    bufr[q],
                    g_hbm.at[pl.ds(tile_base + rbase + q * KG, KG)],
                    wsem))
            for w in ws:
                w.wait()
            return carry2

        lax.fori_loop(0, PWC // (NBUF * KG), rnd, 0)

    return _sc_gather


KX = 400


@functools.cache
def _sc_xgather_call():
    @functools.partial(
        pl.kernel,
        out_type=jax.ShapeDtypeStruct((E, XW), _f32),
        mesh=_mesh(),
        scratch_types=[
            pltpu.VMEM((PER_W,), jnp.int32),
            pltpu.VMEM((PER_W,), jnp.int32),
            [pltpu.VMEM((KX, XW), _f32) for _ in range(NBUF)],
            [pltpu.VMEM((KX, XW), _f32) for _ in range(NBUF)],
            pltpu.SemaphoreType.DMA,
            pltpu.SemaphoreType.DMA,
        ],
        compiler_params=pltpu.CompilerParams(use_tc_tiling_on_sc=False),
    )
    def _sc_xgather(xp_hbm, idx_hbm, rx_hbm,
                    ir_v, ic_v, bufr, bufc, gsem, wsem):
        wid = lax.axis_index("s") * NC + lax.axis_index("c")
        tile_base = wid * PER_W
        pltpu.sync_copy(idx_hbm.at[0, pl.ds(tile_base, PER_W)], ir_v)
        pltpu.sync_copy(idx_hbm.at[1, pl.ds(tile_base, PER_W)], ic_v)

        def rnd(r, carry2):
            rbase = r * (NBUF * KX)
            gs = []
            for q in range(NBUF):
                isl = pl.ds(rbase + q * KX, KX)
                gs.append((
                    pltpu.async_copy(xp_hbm.at[ir_v.at[isl]], bufr[q], gsem),
                    pltpu.async_copy(xp_hbm.at[ic_v.at[isl]], bufc[q], gsem),
                ))
            ws = []
            for q in range(NBUF):
                gs[q][0].wait()
                gs[q][1].wait()

                def sub_row(i, carry3, q=q):
                    bufr[q][i, :] = bufr[q][i, :] - bufc[q][i, :]
                    return carry3

                lax.fori_loop(0, KX, sub_row, 0)
                ws.append(pltpu.async_copy(
                    bufr[q],
                    rx_hbm.at[pl.ds(tile_base + rbase + q * KX, KX)],
                    wsem))
            for w in ws:
                w.wait()
            return carry2

        lax.fori_loop(0, PER_W // (NBUF * KX), rnd, 0)

    return _sc_xgather


def _edge_body(s_ref, sx_ref, ea_ref,
               w1a_ref, w1d_ref, b1_ref, w2_ref, b2_ref, w3_ref, b3_ref,
               aw1_ref, ab1_ref, aw2_ref, ab2_ref,
               cw1_ref, cb1_ref, cw2_ref, cb2_ref, cw3_ref, cb3_ref,
               m_ref, cv_ref):
    silu = jax.nn.silu
    rel = sx_ref[:, :C]
    dist = jnp.sqrt(jnp.sum(rel * rel, axis=-1, keepdims=True) + 1e-12)
    pre1 = (jnp.dot(ea_ref[...], w1a_ref[...], preferred_element_type=_f32)
            + s_ref[...] + dist * w1d_ref[...] + b1_ref[...])
    m = silu(pre1)
    m = silu(jnp.dot(m, w2_ref[...], preferred_element_type=_f32) + b2_ref[...])
    m = jnp.dot(m, w3_ref[...], preferred_element_type=_f32) + b3_ref[...]
    a = silu(jnp.dot(m, aw1_ref[...], preferred_element_type=_f32) + ab1_ref[...])
    att = jax.nn.sigmoid(
        jnp.dot(a, aw2_ref[...], preferred_element_type=_f32) + ab2_ref[...])
    m = m * att
    c = silu(jnp.dot(m, cw1_ref[...], preferred_element_type=_f32) + cb1_ref[...])
    c = silu(jnp.dot(c, cw2_ref[...], preferred_element_type=_f32) + cb2_ref[...])
    co = jnp.dot(c, cw3_ref[...], preferred_element_type=_f32) + cb3_ref[...]
    cvec = co * rel / (dist + 1e-08)
    m_ref[...] = m
    cv_ref[...] = jnp.concatenate(
        [cvec, jnp.zeros((BE, XW - C), _f32)], axis=1)


_full = lambda r, c: pl.BlockSpec((r, c), lambda i: (0, 0))
NBE = EC // BE


@functools.cache
def _edge_call(ci):
    off = ci * NBE
    return pl.pallas_call(
        _edge_body,
        grid=(NBE,),
        in_specs=[
            pl.BlockSpec((BE, D), lambda i: (i, 0)),
            pl.BlockSpec((BE, XW), lambda i: (off + i, 0)),
            pl.BlockSpec((BE, 4), lambda i: (off + i, 0)),
            _full(4, H), _full(1, H), _full(1, H),
            _full(H, H), _full(1, H), _full(H, H), _full(1, H),
            _full(H, H), _full(1, H), _full(H, 1), _full(1, 1),
            _full(H, H), _full(1, H), _full(H, H), _full(1, H),
            _full(H, C), _full(1, C),
        ],
        out_specs=[
            pl.BlockSpec((BE, D), lambda i: (i, 0)),
            pl.BlockSpec((BE, XW), lambda i: (i, 0)),
        ],
        out_shape=[
            jax.ShapeDtypeStruct((EC, D), _f32),
            jax.ShapeDtypeStruct((EC, XW), _f32),
        ],
    )


KSC = 200


@functools.cache
def _sc_scatter_call():
    @functools.partial(
        pl.kernel,
        out_type=jax.ShapeDtypeStruct((NC, N_PAD, D), _f32),
        mesh=_mesh(),
        scratch_types=[
            pltpu.VMEM((KSC,), jnp.int32),
            pltpu.VMEM((KSC, D), _f32),
            pltpu.VMEM_SHARED((N_PAD, D), _f32),
            pltpu.SemaphoreType.DMA,
        ],
        compiler_params=pltpu.CompilerParams(use_tc_tiling_on_sc=True),
    )
    def _sc_scatter(m0, m1, m2, m3, m4, row_hbm, zero_hbm, out_hbm,
                    idx_v, mbuf_v, acc_sh, sem):
        cid = lax.axis_index("c")
        sid = lax.axis_index("s")
        wid = sid * NC + cid
        pltpu.sync_copy(zero_hbm.at[pl.ds(sid * RPT, RPT)],
                        acc_sh.at[pl.ds(sid * RPT, RPT)])
        plsc.subcore_barrier()

        for ci, mc in enumerate((m0, m1, m2, m3, m4)):
            def body(i, carry, mc=mc, ci=ci):
                lbase = wid * PWC + i * KSC
                pltpu.sync_copy(row_hbm.at[pl.ds(ci * EC + lbase, KSC)],
                                idx_v)
                pltpu.sync_copy(mc.at[pl.ds(lbase, KSC)], mbuf_v)
                pltpu.sync_copy(mbuf_v, acc_sh.at[idx_v], add=True)
                return carry

            lax.fori_loop(0, PWC // KSC, body, 0)
        plsc.subcore_barrier()
        pltpu.sync_copy(acc_sh.at[pl.ds(sid * RPT, RPT)],
                        out_hbm.at[cid, pl.ds(sid * RPT, RPT)])

    return _sc_scatter


KXS = 1000


@functools.cache
def _sc_xscatter_call():
    @functools.partial(
        pl.kernel,
        out_type=jax.ShapeDtypeStruct((NC, N_PAD, XW), _f32),
        mesh=_mesh(),
        scratch_types=[
            pltpu.VMEM((KXS,), jnp.int32),
            pltpu.VMEM((KXS, XW), _f32),
            pltpu.VMEM_SHARED((N_PAD, XW), _f32),
            pltpu.SemaphoreType.DMA,
        ],
        compiler_params=pltpu.CompilerParams(use_tc_tiling_on_sc=False),
    )
    def _sc_xscatter(c0, c1, c2, c3, c4, idx_hbm, zero_hbm, out_hbm,
                     idx_v, cbuf_v, acc_sh, sem):
        cid = lax.axis_index("c")
        sid = lax.axis_index("s")
        wid = sid * NC + cid
        pltpu.sync_copy(zero_hbm.at[pl.ds(sid * RPT, RPT)],
                        acc_sh.at[pl.ds(sid * RPT, RPT)])
        plsc.subcore_barrier()

        for ci, cc in enumerate((c0, c1, c2, c3, c4)):
            def body(i, carry, cc=cc, ci=ci):
                lbase = wid * PWC + i * KXS
                pltpu.sync_copy(idx_hbm.at[0, pl.ds(ci * EC + lbase, KXS)],
                                idx_v)
                pltpu.sync_copy(cc.at[pl.ds(lbase, KXS)], cbuf_v)
                pltpu.sync_copy(cbuf_v, acc_sh.at[idx_v], add=True)
                return carry

            lax.fori_loop(0, PWC // KXS, body, 0)
        plsc.subcore_barrier()
        pltpu.sync_copy(acc_sh.at[pl.ds(sid * RPT, RPT)],
                        out_hbm.at[cid, pl.ds(sid * RPT, RPT)])

    return _sc_xscatter


def _node_body(h_ref, x_ref, p_ref, px_ref,
               w1h_ref, w1a_ref, b1_ref, w2_ref, b2_ref, w3_ref, b3_ref,
               ho_ref, xo_ref):
    silu = jax.nn.silu
    h = h_ref[...]
    aggr = p_ref[0] + p_ref[1]
    caggr = px_ref[0, :, :C] + px_ref[1, :, :C]
    t = silu(jnp.dot(h, w1h_ref[...], preferred_element_type=_f32)
             + jnp.dot(aggr, w1a_ref[...], preferred_element_type=_f32)
             + b1_ref[...])
    t = silu(jnp.dot(t, w2_ref[...], preferred_element_type=_f32) + b2_ref[...])
    ho_ref[...] = h + jnp.dot(t, w3_ref[...],
                              preferred_element_type=_f32) + b3_ref[...]
    xo_ref[...] = x_ref[...] + caggr


_node_call = pl.pallas_call(
    _node_body,
    grid=(N // BN,),
    in_specs=[
        pl.BlockSpec((BN, D), lambda i: (i, 0)),
        pl.BlockSpec((BN, C), lambda i: (i, 0)),
        pl.BlockSpec((NC, BN, D), lambda i: (0, i, 0)),
        pl.BlockSpec((NC, BN, XW), lambda i: (0, i, 0)),
        _full(D, H), _full(H, H), _full(1, H),
        _full(H, H), _full(1, H), _full(H, D), _full(1, D),
    ],
    out_specs=[
        pl.BlockSpec((BN, D), lambda i: (i, 0)),
        pl.BlockSpec((BN, C), lambda i: (i, 0)),
    ],
    out_shape=[
        jax.ShapeDtypeStruct((N, D), _f32),
        jax.ShapeDtypeStruct((N, C), _f32),
    ],
)


def kernel(h, x, edge_index, edge_attr,
           ew1, eb1, ew2, eb2, ew3, eb3,
           nw1, nb1, nw2, nb2, nw3, nb3,
           cw1, cb1, cw2, cb2, cw3, cb3,
           aw1, ab1, aw2, ab2):
    eidx = edge_index.astype(jnp.int32)
    row = eidx[0]
    col = eidx[1]
    xp = jnp.pad(x, ((0, 0), (0, XW - C)))

    t_r, t_c = _table_call(h, ew1[4:4 + D], ew1[4 + D:4 + 2 * D])
    relx = _sc_xgather_call()(xp, eidx)
    edge_w = (ew1[:4], ew1[4 + 2 * D:].reshape(1, H), eb1.reshape(1, H),
              ew2, eb2.reshape(1, H), ew3, eb3.reshape(1, H),
              aw1, ab1.reshape(1, H), aw2, ab2.reshape(1, 1),
              cw1, cb1.reshape(1, H), cw2, cb2.reshape(1, H),
              cw3, cb3.reshape(1, C))
    ms, cvs = [], []
    for ci in range(NCHK):
        s_c = _sc_gather_call(ci)(t_r, t_c, row, col)
        m_c, cv_c = _edge_call(ci)(s_c, relx, edge_attr, *edge_w)
        ms.append(m_c)
        cvs.append(cv_c)
    zeros_m = jnp.zeros((N_PAD, D), _f32)
    zeros_x = jnp.zeros((N_PAD, XW), _f32)
    partials = _sc_scatter_call()(*ms, row, zeros_m)
    partials_x = _sc_xscatter_call()(*cvs, eidx, zeros_x)
    h_out, x_out = _node_call(
        h, x, partials, partials_x,
        nw1[:D], nw1[D:], nb1.reshape(1, H),
        nw2, nb2.reshape(1, H), nw3, nb3.reshape(1, D))
    return (h_out, x_out)


# speedup vs baseline: 1.1560x; 1.0173x over previous
import functools

import jax
import jax.numpy as jnp
from jax import lax
from jax.experimental import pallas as pl
from jax.experimental.pallas import tpu as pltpu
from jax.experimental.pallas import tpu_sc as plsc

N = 10000
E = 320000
D = 128
H = 128
C = 3
XW = 16
N_PAD = 10240

NC = 2
NS = 16
NW = NC * NS
PER_W = E // NW
RPT = N_PAD // NS

NCHK = 5
EC = E // NCHK
PWC = EC // NW

BN = 1000
BE = 4000

_f32 = jnp.float32


def _mesh():
    return plsc.VectorSubcoreMesh(core_axis_name="c", subcore_axis_name="s",
                                  num_cores=NC, num_subcores=NS)


def _table_body(h_ref, wr_ref, wc_ref, tr_ref, tc_ref):
    h = h_ref[...]
    tr_ref[...] = jnp.dot(h, wr_ref[...], preferred_element_type=_f32)
    tc_ref[...] = jnp.dot(h, wc_ref[...], preferred_element_type=_f32)


_table_call = pl.pallas_call(
    _table_body,
    grid=(N // BN,),
    in_specs=[
        pl.BlockSpec((BN, D), lambda i: (i, 0)),
        pl.BlockSpec((D, D), lambda i: (0, 0)),
        pl.BlockSpec((D, D), lambda i: (0, 0)),
    ],
    out_specs=[
        pl.BlockSpec((BN, D), lambda i: (i, 0)),
        pl.BlockSpec((BN, D), lambda i: (i, 0)),
    ],
    out_shape=[
        jax.ShapeDtypeStruct((N, D), _f32),
        jax.ShapeDtypeStruct((N, D), _f32),
    ],
)


KG = 80
NBUF = 5


@functools.cache
def _sc_gather_call(ci):
    @functools.partial(
        pl.kernel,
        out_type=jax.ShapeDtypeStruct((EC, D), _f32),
        mesh=_mesh(),
        scratch_types=[
            pltpu.VMEM((PWC,), jnp.int32),
            pltpu.VMEM((PWC,), jnp.int32),
            [pltpu.VMEM((KG, D), _f32) for _ in range(NBUF)],
            [pltpu.VMEM((KG, D), _f32) for _ in range(NBUF)],
            pltpu.SemaphoreType.DMA,
            pltpu.SemaphoreType.DMA,
        ],
        compiler_params=pltpu.CompilerParams(use_tc_tiling_on_sc=True),
    )
    def _sc_gather(tr_hbm, tc_hbm, row_hbm, col_hbm, g_hbm,
                   ir_v, ic_v, bufr, bufc, gsem, wsem):
        wid = lax.axis_index("s") * NC + lax.axis_index("c")
        tile_base = wid * PWC
        pltpu.sync_copy(row_hbm.at[pl.ds(ci * EC + tile_base, PWC)], ir_v)
        pltpu.sync_copy(col_hbm.at[pl.ds(ci * EC + tile_base, PWC)], ic_v)

        def rnd(r, carry2):
            rbase = r * (NBUF * KG)
            gs = []
            for q in range(NBUF):
                isl = pl.ds(rbase + q * KG, KG)
                gs.append((
                    pltpu.async_copy(tr_hbm.at[ir_v.at[isl]], bufr[q], gsem),
                    pltpu.async_copy(tc_hbm.at[ic_v.at[isl]], bufc[q], gsem),
                ))
            ws = []
            for q in range(NBUF):
                gs[q][0].wait()
                gs[q][1].wait()

                def add_row(i, carry3, q=q):
                    for j in range(D // 16):
                        sl = pl.ds(j * 16, 16)
                        bufr[q][i, sl] = bufr[q][i, sl] + bufc[q][i, sl]
                    return carry3

                lax.fori_loop(0, KG, add_row, 0)
                ws.append(pltpu.async_copy(
                    bufr[q],
                    g_hbm.at[pl.ds(tile_base + rbase + q * KG, KG)],
                    wsem))
            for w in ws:
                w.wait()
            return carry2

        lax.fori_loop(0, PWC // (NBUF * KG), rnd, 0)

    return _sc_gather


KX = 400


@functools.cache
def _sc_xgather_call():
    @functools.partial(
        pl.kernel,
        out_type=jax.ShapeDtypeStruct((E, XW), _f32),
        mesh=_mesh(),
        scratch_types=[
            pltpu.VMEM((PER_W,), jnp.int32),
            pltpu.VMEM((PER_W,), jnp.int32),
            [pltpu.VMEM((KX, XW), _f32) for _ in range(NBUF)],
            [pltpu.VMEM((KX, XW), _f32) for _ in range(NBUF)],
            pltpu.SemaphoreType.DMA,
            pltpu.SemaphoreType.DMA,
        ],
        compiler_params=pltpu.CompilerParams(use_tc_tiling_on_sc=False),
    )
    def _sc_xgather(xp_hbm, idx_hbm, rx_hbm,
                    ir_v, ic_v, bufr, bufc, gsem, wsem):
        wid = lax.axis_index("s") * NC + lax.axis_index("c")
        tile_base = wid * PER_W
        pltpu.sync_copy(idx_hbm.at[0, pl.ds(tile_base, PER_W)], ir_v)
        pltpu.sync_copy(idx_hbm.at[1, pl.ds(tile_base, PER_W)], ic_v)

        def rnd(r, carry2):
            rbase = r * (NBUF * KX)
            gs = []
            for q in range(NBUF):
                isl = pl.ds(rbase + q * KX, KX)
                gs.append((
                    pltpu.async_copy(xp_hbm.at[ir_v.at[isl]], bufr[q], gsem),
                    pltpu.async_copy(xp_hbm.at[ic_v.at[isl]], bufc[q], gsem),
                ))
            ws = []
            for q in range(NBUF):
                gs[q][0].wait()
                gs[q][1].wait()

                def sub_row(i, carry3, q=q):
                    bufr[q][i, :] = bufr[q][i, :] - bufc[q][i, :]
                    return carry3

                lax.fori_loop(0, KX, sub_row, 0)
                ws.append(pltpu.async_copy(
                    bufr[q],
                    rx_hbm.at[pl.ds(tile_base + rbase + q * KX, KX)],
                    wsem))
            for w in ws:
                w.wait()
            return carry2

        lax.fori_loop(0, PER_W // (NBUF * KX), rnd, 0)

    return _sc_xgather


def _edge_body(s_ref, sx_ref, ea_ref,
               w1a_ref, w1d_ref, b1_ref, w2_ref, b2_ref, w3_ref, b3_ref,
               aw1_ref, ab1_ref, aw2_ref, ab2_ref,
               cw1_ref, cb1_ref, cw2_ref, cb2_ref, cw3_ref, cb3_ref,
               m_ref, cv_ref):
    silu = jax.nn.silu
    rel = sx_ref[:, :C]
    dist = jnp.sqrt(jnp.sum(rel * rel, axis=-1, keepdims=True) + 1e-12)
    pre1 = (jnp.dot(ea_ref[...], w1a_ref[...], preferred_element_type=_f32)
            + s_ref[...] + dist * w1d_ref[...] + b1_ref[...])
    m = silu(pre1)
    m = silu(jnp.dot(m, w2_ref[...], preferred_element_type=_f32) + b2_ref[...])
    m = jnp.dot(m, w3_ref[...], preferred_element_type=_f32) + b3_ref[...]
    a = silu(jnp.dot(m, aw1_ref[...], preferred_element_type=_f32) + ab1_ref[...])
    att = jax.nn.sigmoid(
        jnp.dot(a, aw2_ref[...], preferred_element_type=_f32) + ab2_ref[...])
    m = m * att
    c = silu(jnp.dot(m, cw1_ref[...], preferred_element_type=_f32) + cb1_ref[...])
    c = silu(jnp.dot(c, cw2_ref[...], preferred_element_type=_f32) + cb2_ref[...])
    co = jnp.dot(c, cw3_ref[...], preferred_element_type=_f32) + cb3_ref[...]
    cvec = co * rel / (dist + 1e-08)
    m_ref[...] = m
    cv_ref[...] = jnp.concatenate(
        [cvec, jnp.zeros((BE, XW - C), _f32)], axis=1)


_full = lambda r, c: pl.BlockSpec((r, c), lambda i: (0, 0))
NBE = EC // BE


@functools.cache
def _edge_call(ci):
    off = ci * NBE
    return pl.pallas_call(
        _edge_body,
        grid=(NBE,),
        in_specs=[
            pl.BlockSpec((BE, D), lambda i: (i, 0)),
            pl.BlockSpec((BE, XW), lambda i: (off + i, 0)),
            pl.BlockSpec((BE, 4), lambda i: (off + i, 0)),
            _full(4, H), _full(1, H), _full(1, H),
            _full(H, H), _full(1, H), _full(H, H), _full(1, H),
            _full(H, H), _full(1, H), _full(H, 1), _full(1, 1),
            _full(H, H), _full(1, H), _full(H, H), _full(1, H),
            _full(H, C), _full(1, C),
        ],
        out_specs=[
            pl.BlockSpec((BE, D), lambda i: (i, 0)),
            pl.BlockSpec((BE, XW), lambda i: (i, 0)),
        ],
        out_shape=[
            jax.ShapeDtypeStruct((EC, D), _f32),
            jax.ShapeDtypeStruct((EC, XW), _f32),
        ],
    )


KSC = 200


@functools.cache
def _sc_scatter_call():
    @functools.partial(
        pl.kernel,
        out_type=jax.ShapeDtypeStruct((NC, N_PAD, D), _f32),
        mesh=_mesh(),
        scratch_types=[
            pltpu.VMEM((KSC,), jnp.int32),
            pltpu.VMEM((KSC, D), _f32),
            pltpu.VMEM_SHARED((N_PAD, D), _f32),
            pltpu.SemaphoreType.DMA,
        ],
        compiler_params=pltpu.CompilerParams(use_tc_tiling_on_sc=True),
    )
    def _sc_scatter(m0, m1, m2, m3, m4, row_hbm, zero_hbm, out_hbm,
                    idx_v, mbuf_v, acc_sh, sem):
        cid = lax.axis_index("c")
        sid = lax.axis_index("s")
        wid = sid * NC + cid
        pltpu.sync_copy(zero_hbm.at[pl.ds(sid * RPT, RPT)],
                        acc_sh.at[pl.ds(sid * RPT, RPT)])
        plsc.subcore_barrier()

        for ci, mc in enumerate((m0, m1, m2, m3, m4)):
            def body(i, carry, mc=mc, ci=ci):
                lbase = wid * PWC + i * KSC
                pltpu.sync_copy(row_hbm.at[pl.ds(ci * EC + lbase, KSC)],
                                idx_v)
                pltpu.sync_copy(mc.at[pl.ds(lbase, KSC)], mbuf_v)
                pltpu.sync_copy(mbuf_v, acc_sh.at[idx_v], add=True)
                return carry

            lax.fori_loop(0, PWC // KSC, body, 0)
        plsc.subcore_barrier()
        pltpu.sync_copy(acc_sh.at[pl.ds(sid * RPT, RPT)],
                        out_hbm.at[cid, pl.ds(sid * RPT, RPT)])

    return _sc_scatter


KXS = 1000


@functools.cache
def _sc_xscatter_call():
    @functools.partial(
        pl.kernel,
        out_type=jax.ShapeDtypeStruct((NC, N_PAD, XW), _f32),
        mesh=_mesh(),
        scratch_types=[
            pltpu.VMEM((KXS,), jnp.int32),
            pltpu.VMEM((KXS, XW), _f32),
            pltpu.VMEM_SHARED((N_PAD, XW), _f32),
            pltpu.SemaphoreType.DMA,
        ],
        compiler_params=pltpu.CompilerParams(use_tc_tiling_on_sc=False),
    )
    def _sc_xscatter(c0, c1, c2, c3, c4, idx_hbm, zero_hbm, out_hbm,
                     idx_v, cbuf_v, acc_sh, sem):
        cid = lax.axis_index("c")
        sid = lax.axis_index("s")
        wid = sid * NC + cid
        pltpu.sync_copy(zero_hbm.at[pl.ds(sid * RPT, RPT)],
                        acc_sh.at[pl.ds(sid * RPT, RPT)])
        plsc.subcore_barrier()

        for ci, cc in enumerate((c0, c1, c2, c3, c4)):
            def body(i, carry, cc=cc, ci=ci):
                lbase = wid * PWC + i * KXS
                pltpu.sync_copy(idx_hbm.at[0, pl.ds(ci * EC + lbase, KXS)],
                                idx_v)
                pltpu.sync_copy(cc.at[pl.ds(lbase, KXS)], cbuf_v)
                pltpu.sync_copy(cbuf_v, acc_sh.at[idx_v], add=True)
                return carry

            lax.fori_loop(0, PWC // KXS, body, 0)
        plsc.subcore_barrier()
        pltpu.sync_copy(acc_sh.at[pl.ds(sid * RPT, RPT)],
                        out_hbm.at[cid, pl.ds(sid * RPT, RPT)])

    return _sc_xscatter


def _node_body(h_ref, x_ref, p_ref, px_ref,
               w1h_ref, w1a_ref, b1_ref, w2_ref, b2_ref, w3_ref, b3_ref,
               ho_ref, xo_ref):
    silu = jax.nn.silu
    h = h_ref[...]
    aggr = p_ref[0] + p_ref[1]
    caggr = px_ref[0, :, :C] + px_ref[1, :, :C]
    t = silu(jnp.dot(h, w1h_ref[...], preferred_element_type=_f32)
             + jnp.dot(aggr, w1a_ref[...], preferred_element_type=_f32)
             + b1_ref[...])
    t = silu(jnp.dot(t, w2_ref[...], preferred_element_type=_f32) + b2_ref[...])
    ho_ref[...] = h + jnp.dot(t, w3_ref[...],
                              preferred_element_type=_f32) + b3_ref[...]
    xo_ref[...] = x_ref[...] + caggr


_node_call = pl.pallas_call(
    _node_body,
    grid=(N // BN,),
    in_specs=[
        pl.BlockSpec((BN, D), lambda i: (i, 0)),
        pl.BlockSpec((BN, C), lambda i: (i, 0)),
        pl.BlockSpec((NC, BN, D), lambda i: (0, i, 0)),
        pl.BlockSpec((NC, BN, XW), lambda i: (0, i, 0)),
        _full(D, H), _full(H, H), _full(1, H),
        _full(H, H), _full(1, H), _full(H, D), _full(1, D),
    ],
    out_specs=[
        pl.BlockSpec((BN, D), lambda i: (i, 0)),
        pl.BlockSpec((BN, C), lambda i: (i, 0)),
    ],
    out_shape=[
        jax.ShapeDtypeStruct((N, D), _f32),
        jax.ShapeDtypeStruct((N, C), _f32),
    ],
)


def kernel(h, x, edge_index, edge_attr,
           ew1, eb1, ew2, eb2, ew3, eb3,
           nw1, nb1, nw2, nb2, nw3, nb3,
           cw1, cb1, cw2, cb2, cw3, cb3,
           aw1, ab1, aw2, ab2):
    eidx = edge_index.astype(jnp.int32)
    row = eidx[0]
    col = eidx[1]
    xp = jnp.pad(x, ((0, 0), (0, XW - C)))

    t_r, t_c = _table_call(h, ew1[4:4 + D], ew1[4 + D:4 + 2 * D])
    relx = _sc_xgather_call()(xp, eidx)
    edge_w = (ew1[:4], ew1[4 + 2 * D:].reshape(1, H), eb1.reshape(1, H),
              ew2, eb2.reshape(1, H), ew3, eb3.reshape(1, H),
              aw1, ab1.reshape(1, H), aw2, ab2.reshape(1, 1),
              cw1, cb1.reshape(1, H), cw2, cb2.reshape(1, H),
              cw3, cb3.reshape(1, C))
    ms, cvs = [], []
    for ci in range(NCHK):
        s_c = _sc_gather_call(ci)(t_r, t_c, row, col)
        m_c, cv_c = _edge_call(ci)(s_c, relx, edge_attr, *edge_w)
        ms.append(m_c)
        cvs.append(cv_c)
    zeros_m = jnp.zeros((N_PAD, D), _f32)
    zeros_x = jnp.zeros((N_PAD, XW), _f32)
    partials = _sc_scatter_call()(*ms, row, zeros_m)
    partials_x = _sc_xscatter_call()(*cvs, eidx, zeros_x)
    h_out, x_out = _node_call(
        h, x, partials, partials_x,
        nw1[:D], nw1[D:], nb1.reshape(1, H),
        nw2, nb2.reshape(1, H), nw3, nb3.reshape(1, D))
    return (h_out, x_out)
